# baseline (speedup 1.0000x reference)
import jax
import jax.numpy as jnp
from jax.experimental import pallas as pl

N, T, P, D = 16, 288, 325, 64
TB = 24


def _body(x_ref, m_ref, wt_ref, bt_ref, wv_ref, bv_ref, et_ref, ut_ref, o_ref):
    xb = x_ref[...]
    v = xb[..., 0:1]
    t = xb[..., 1:2]
    m = m_ref[...]
    inv = jnp.isnan(v)
    safe = jnp.where(inv, 0.0, v)
    code = m * 2.0 + inv.astype(jnp.float32)
    wt = wt_ref[...].reshape(1, 1, 1, D)
    bt = bt_ref[...].reshape(1, 1, 1, D)
    wv = wv_ref[...].reshape(1, 1, 1, D)
    bv = bv_ref[...].reshape(1, 1, 1, D)
    et = et_ref[...].reshape(1, 1, 1, D)
    ut = ut_ref[...].reshape(1, 1, 1, D)
    time_emb = t * wt + bt
    val_emb = safe * wv + bv
    val_emb = jnp.where(code < 2.0, ut, val_emb)
    val_emb = jnp.where(code == 3.0, et, val_emb)
    o_ref[...] = time_emb + val_emb


def kernel(x, monitor_mask, time_emb_w, time_emb_b, value_emb_w, value_emb_b,
           empty_token, unmonitored_token):
    mf = monitor_mask.astype(jnp.float32)[..., None]
    w_spec = pl.BlockSpec((1, D), lambda i, j: (0, 0))
    out = pl.pallas_call(
        _body,
        grid=(N, T // TB),
        in_specs=[pl.BlockSpec((1, TB, P, 2), lambda i, j: (i, j, 0, 0)),
                  pl.BlockSpec((1, TB, P, 1), lambda i, j: (i, j, 0, 0))] + [w_spec] * 6,
        out_specs=pl.BlockSpec((1, TB, P, D), lambda i, j: (i, j, 0, 0)),
        out_shape=jax.ShapeDtypeStruct((N, T, P, D), jnp.float32),
    )(x, mf, time_emb_w, time_emb_b, value_emb_w, value_emb_b,
      empty_token.reshape(1, D), unmonitored_token.reshape(1, D))
    return out


# speedup vs baseline: 1.2657x; 1.2657x over previous
import functools
import jax
import jax.numpy as jnp
from jax import lax
from jax.experimental import pallas as pl
from jax.experimental.pallas import tpu as pltpu
from jax.experimental.pallas import tpu_sc as plsc

N, T, P, D = 16, 288, 325, 64
R = N * T * P
NW = 32
RW = R // NW
C = 1200
NCH = RW // C

_mesh = plsc.VectorSubcoreMesh(core_axis_name="c", subcore_axis_name="s")


@functools.partial(
    pl.kernel, mesh=_mesh,
    out_type=jax.ShapeDtypeStruct((R * D,), jnp.float32),
    scratch_types=[
        pltpu.VMEM((C + 16,), jnp.float32),
        pltpu.VMEM((C + 16,), jnp.float32),
        pltpu.VMEM((C + 16,), jnp.float32),
        pltpu.VMEM((C + 16,), jnp.float32),
        pltpu.VMEM((C * D,), jnp.float32),
        pltpu.VMEM((5 * D,), jnp.float32),
    ],
)
def _sc_embed(vf, tf, mf, wf, out_hbm, vv, tv, mv, s2v, ov, wvm):
    wid = lax.axis_index("s") * 2 + lax.axis_index("c")
    base0 = wid * RW
    pltpu.sync_copy(wf, wvm.at[pl.ds(0, 5 * D)])
    wt = [wvm[pl.ds(16 * j, 16)] for j in range(4)]
    wv4 = [wvm[pl.ds(D + 16 * j, 16)] for j in range(4)]
    a2 = [wvm[pl.ds(2 * D + 16 * j, 16)] for j in range(4)]
    a3 = [wvm[pl.ds(3 * D + 16 * j, 16)] for j in range(4)]
    c1 = [wvm[pl.ds(4 * D + 16 * j, 16)] for j in range(4)]

    def chunk_body(ci, carry):
        base = base0 + ci * C
        pltpu.sync_copy(vf.at[pl.ds(base, C)], vv.at[pl.ds(0, C)])
        pltpu.sync_copy(tf.at[pl.ds(base, C)], tv.at[pl.ds(0, C)])
        pltpu.sync_copy(mf.at[pl.ds(base, C)], mv.at[pl.ds(0, C)])

        def coeff_body(g, carry2):
            sl = pl.ds(g * 16, 16)
            v16 = vv[sl]
            m16 = mv[sl]
            inv = jnp.isnan(v16)
            invf = jnp.where(inv, 1.0, 0.0)
            vv[sl] = jnp.where(inv, 0.0, v16) * m16
            s2v[sl] = 1.0 - m16
            mv[sl] = m16 * invf
            return carry2

        lax.fori_loop(0, C // 16, coeff_body, 0)

        def row_body(r, carry2):
            t_s = tv[pl.ds(r, 16)][0]
            u_s = vv[pl.ds(r, 16)][0]
            s2_s = s2v[pl.ds(r, 16)][0]
            s3_s = mv[pl.ds(r, 16)][0]
            for j in range(4):
                acc = (c1[j] + t_s * wt[j] + u_s * wv4[j]
                       + s2_s * a2[j] + s3_s * a3[j])
                ov[pl.ds(r * 64 + 16 * j, 16)] = acc
            return carry2

        lax.fori_loop(0, C, row_body, 0)
        pltpu.sync_copy(ov, out_hbm.at[pl.ds(base * 64, C * 64)])
        return carry

    lax.fori_loop(0, NCH, chunk_body, 0)


def kernel(x, monitor_mask, time_emb_w, time_emb_b, value_emb_w, value_emb_b,
           empty_token, unmonitored_token):
    vf = x[..., 0].reshape(R)
    tf = x[..., 1].reshape(R)
    mf = monitor_mask.astype(jnp.float32).reshape(R)
    wt = time_emb_w.reshape(D)
    wv = value_emb_w.reshape(D)
    bt = time_emb_b.reshape(D)
    bv = value_emb_b.reshape(D)
    a2 = unmonitored_token - bv
    a3 = empty_token - bv
    c1 = bt + bv
    wf = jnp.concatenate([wt, wv, a2, a3, c1])
    out = _sc_embed(vf, tf, mf, wf)
    return out.reshape(N, T, P, D)
